# baseline (speedup 1.0000x reference)
import jax
import jax.numpy as jnp
from jax.experimental import pallas as pl
from jax.experimental.pallas import tpu as pltpu

_B = 16
_N = 32768
_NF = 32
_C1 = 64
_C2 = 16
_K = 64
_CN = 4096
_EPS = 1e-5
_BIGI = 2 ** 30


def _first_step():
    return (pl.program_id(0) == 0) & (pl.program_id(1) == 0)


def _stats_cols(x):
    ssum = jnp.sum(x, axis=1, keepdims=True)
    ssq = jnp.sum(x * x, axis=1, keepdims=True)
    lane = jax.lax.broadcasted_iota(jnp.int32, (x.shape[0], 128), 1)
    return jnp.where(lane == 0, ssum, 0.0) + jnp.where(lane == 1, ssq, 0.0)


def _mm(w, x):
    return jax.lax.dot_general(w, x, (((1,), (0,)), ((), ())),
                               preferred_element_type=jnp.float32)


def _stats_kernel_a(s_ref, w_ref, b_ref, out_ref):
    x = _mm(w_ref[...], s_ref[0].astype(jnp.bfloat16)) + b_ref[...]

    @pl.when(_first_step())
    def _():
        out_ref[...] = jnp.zeros_like(out_ref)

    out_ref[...] += _stats_cols(x)


def _layer12(s_ref, w0_ref, aff0_ref, w1_ref, b1_ref):
    x0 = _mm(w0_ref[...], s_ref[0].astype(jnp.bfloat16)) + aff0_ref[:, 2:3]
    x1 = jnp.maximum(aff0_ref[:, 0:1] * x0 + aff0_ref[:, 1:2], 0.0)
    return _mm(w1_ref[...], x1.astype(jnp.bfloat16)) + b1_ref[...]


def _stats_kernel_b(s_ref, w0_ref, aff0_ref, w1_ref, b1_ref, out_ref):
    t = _layer12(s_ref, w0_ref, aff0_ref, w1_ref, b1_ref)

    @pl.when(_first_step())
    def _():
        out_ref[...] = jnp.zeros_like(out_ref)

    out_ref[...] += _stats_cols(t)


def _score_kernel_c(s_ref, w0_ref, aff0_ref, w1_ref, b1_ref, aff1_ref,
                    w2_ref, b2_ref, u_ref):
    t = _layer12(s_ref, w0_ref, aff0_ref, w1_ref, b1_ref)
    x2 = jnp.maximum(aff1_ref[:, 0:1] * t + aff1_ref[:, 1:2], 0.0)
    u = _mm(w2_ref[...], x2.astype(jnp.bfloat16))
    u_ref[0, 0, :] = u[0] + b2_ref[0]


def _topk_kernel(u_ref, gb_ref, vals_ref, idx_ref, work):
    u = u_ref[:, 0, :]
    nn = jnp.float32(_B * _N)
    m2 = jnp.sum(u) / nn
    v2 = jnp.sum(u * u) / nn - m2 * m2
    a2 = gb_ref[0] * jax.lax.rsqrt(v2 + _EPS)
    d2 = gb_ref[1] - a2 * m2
    sv = jnp.maximum(a2 * u + d2, 0.0)
    work[...] = sv

    pos = jax.lax.broadcasted_iota(jnp.int32, (_B, _N), 1)
    kl = jax.lax.broadcasted_iota(jnp.int32, (_B, _K), 1)
    vacc = jnp.zeros((_B, _K), jnp.float32)
    iacc = jnp.zeros((_B, _K), jnp.int32)
    for k in range(_K):
        w = work[...]
        m = jnp.max(w, axis=1, keepdims=True)
        cand = jnp.where(w == m, pos, _BIGI)
        j = jnp.min(cand, axis=1, keepdims=True)
        vacc = vacc + jnp.where(kl == k, m, 0.0)
        iacc = iacc + jnp.where(kl == k, j, 0)
        work[...] = jnp.where(pos == j, -jnp.inf, w)
    vals_ref[...] = vacc
    idx_ref[...] = iacc


def _gather_kernel(idx_sref, in_ref, out_ref):
    b = pl.program_id(0)
    k = pl.program_id(1)
    col = jax.lax.rem(idx_sref[b * _K + k], 128)
    lane = jax.lax.broadcasted_iota(jnp.int32, (_C1, 128), 1)
    sel = jnp.where(lane == col, in_ref[0], 0.0)
    out_ref[0, 0, :] = jnp.sum(sel, axis=1)


def _fold_bn(ssum, ssq, g, be, eps=_EPS):
    nn = jnp.float32(_B * _N)
    m = ssum / nn
    v = ssq / nn - m * m
    a = g * jax.lax.rsqrt(v + eps)
    d = be - a * m
    return a, d


@jax.jit
def kernel(sortvec, input, W0, b0, W1, b1, W2, b2, g0, be0, g1, be1, g2, be2):
    s = sortvec[:, 0, :, :]
    w0b = W0[:, 0, :, 0].astype(jnp.bfloat16)
    w1b = W1[:, :, 0, 0].astype(jnp.bfloat16)
    w2b = jnp.zeros((8, _C2), jnp.bfloat16).at[0].set(
        W2[:, :, 0, 0][0].astype(jnp.bfloat16))
    b0c = b0[:, None]
    b1c = b1[:, None]

    seq2 = pltpu.CompilerParams(dimension_semantics=("arbitrary", "arbitrary"))
    nchunks = _N // _CN
    s_spec = pl.BlockSpec((1, _NF, _CN), lambda b, c: (b, 0, c))
    full = lambda shp: pl.BlockSpec(shp, lambda b, c: tuple(0 for _ in shp))

    stats1 = pl.pallas_call(
        _stats_kernel_a,
        grid=(_B, nchunks),
        in_specs=[s_spec, full((_C1, _NF)), full((_C1, 1))],
        out_specs=full((_C1, 128)),
        out_shape=jax.ShapeDtypeStruct((_C1, 128), jnp.float32),
        compiler_params=seq2,
    )(s, w0b, b0c)
    a0, d0 = _fold_bn(stats1[:, 0], stats1[:, 1], g0, be0)
    aff0 = jnp.stack([a0, d0, b0], axis=1)

    stats2 = pl.pallas_call(
        _stats_kernel_b,
        grid=(_B, nchunks),
        in_specs=[s_spec, full((_C1, _NF)), full((_C1, 3)),
                  full((_C2, _C1)), full((_C2, 1))],
        out_specs=full((_C2, 128)),
        out_shape=jax.ShapeDtypeStruct((_C2, 128), jnp.float32),
        compiler_params=seq2,
    )(s, w0b, aff0, w1b, b1c)
    a1, d1 = _fold_bn(stats2[:, 0], stats2[:, 1], g1, be1)
    aff1 = jnp.stack([a1, d1, b1], axis=1)

    u = pl.pallas_call(
        _score_kernel_c,
        grid=(_B, nchunks),
        in_specs=[s_spec, full((_C1, _NF)), full((_C1, 3)),
                  full((_C2, _C1)), full((_C2, 1)), full((_C2, 3)),
                  full((8, _C2)), pl.BlockSpec(memory_space=pltpu.SMEM)],
        out_specs=pl.BlockSpec((1, 1, _CN), lambda b, c: (b, 0, c)),
        out_shape=jax.ShapeDtypeStruct((_B, 1, _N), jnp.float32),
        compiler_params=seq2,
    )(s, w0b, aff0, w1b, b1c, aff1, w2b, b2)

    gb = jnp.stack([g2[0], be2[0]])
    vals, idx = pl.pallas_call(
        _topk_kernel,
        in_specs=[pl.BlockSpec(memory_space=pltpu.VMEM),
                  pl.BlockSpec(memory_space=pltpu.SMEM)],
        out_specs=[pl.BlockSpec(memory_space=pltpu.VMEM),
                   pl.BlockSpec(memory_space=pltpu.VMEM)],
        out_shape=[jax.ShapeDtypeStruct((_B, _K), jnp.float32),
                   jax.ShapeDtypeStruct((_B, _K), jnp.int32)],
        scratch_shapes=[pltpu.VMEM((_B, _N), jnp.float32)],
    )(u, gb)

    grid_spec = pltpu.PrefetchScalarGridSpec(
        num_scalar_prefetch=1,
        grid=(_B, _K),
        in_specs=[pl.BlockSpec((1, _C1, 128),
                               lambda b, k, iref: (b, 0, iref[b * _K + k] // 128))],
        out_specs=pl.BlockSpec((1, 1, _C1), lambda b, k, iref: (b * _K + k, 0, 0)),
    )
    gathered = pl.pallas_call(
        _gather_kernel,
        grid_spec=grid_spec,
        out_shape=jax.ShapeDtypeStruct((_B * _K, 1, _C1), jnp.float32),
    )(idx.reshape(-1), input)

    sorted_input = jnp.transpose(gathered.reshape(_B, _K, _C1), (0, 2, 1))
    feat = jnp.concatenate([sorted_input, vals[:, None, :]], axis=1)
    return (feat, idx)


# speedup vs baseline: 1.8449x; 1.8449x over previous
import jax
import jax.numpy as jnp
from jax.experimental import pallas as pl
from jax.experimental.pallas import tpu as pltpu

_B = 16
_N = 32768
_NF = 32
_C1 = 64
_C2 = 16
_K = 64
_CN = 4096
_EPS = 1e-5
_BIGI = 2 ** 30


def _first_step():
    return (pl.program_id(0) == 0) & (pl.program_id(1) == 0)


def _stats_cols(x):
    ssum = jnp.sum(x, axis=1, keepdims=True)
    ssq = jnp.sum(x * x, axis=1, keepdims=True)
    lane = jax.lax.broadcasted_iota(jnp.int32, (x.shape[0], 128), 1)
    return jnp.where(lane == 0, ssum, 0.0) + jnp.where(lane == 1, ssq, 0.0)


def _mm(w, x):
    return jax.lax.dot_general(w, x, (((1,), (0,)), ((), ())),
                               preferred_element_type=jnp.float32)


def _stats_kernel_a(s_ref, w_ref, b_ref, out_ref, sbf_ref):
    sb = s_ref[0].astype(jnp.bfloat16)
    sbf_ref[0] = sb
    x = _mm(w_ref[...], sb) + b_ref[...]

    @pl.when(_first_step())
    def _():
        out_ref[...] = jnp.zeros_like(out_ref)

    out_ref[...] += _stats_cols(x)


def _layer12(s_ref, w0_ref, aff0_ref, w1_ref, b1_ref):
    x0 = _mm(w0_ref[...], s_ref[0]) + aff0_ref[:, 2:3]
    x1 = jnp.maximum(aff0_ref[:, 0:1] * x0 + aff0_ref[:, 1:2], 0.0)
    return _mm(w1_ref[...], x1.astype(jnp.bfloat16)) + b1_ref[...]


def _stats_kernel_b(s_ref, w0_ref, aff0_ref, w1_ref, b1_ref, out_ref):
    t = _layer12(s_ref, w0_ref, aff0_ref, w1_ref, b1_ref)

    @pl.when(_first_step())
    def _():
        out_ref[...] = jnp.zeros_like(out_ref)

    out_ref[...] += _stats_cols(t)


def _score_kernel_c(s_ref, w0_ref, aff0_ref, w1_ref, b1_ref, aff1_ref,
                    w2_ref, b2_ref, u_ref):
    t = _layer12(s_ref, w0_ref, aff0_ref, w1_ref, b1_ref)
    x2 = jnp.maximum(aff1_ref[:, 0:1] * t + aff1_ref[:, 1:2], 0.0)
    u = _mm(w2_ref[...], x2.astype(jnp.bfloat16))
    u_ref[0, 0, :] = u[0] + b2_ref[0]


def _topk_kernel(u_ref, gb_ref, vals_ref, idx_ref, work):
    u = u_ref[:, 0, :]
    nn = jnp.float32(_B * _N)
    m2 = jnp.sum(u) / nn
    v2 = jnp.sum(u * u) / nn - m2 * m2
    a2 = gb_ref[0] * jax.lax.rsqrt(v2 + _EPS)
    d2 = gb_ref[1] - a2 * m2
    sv = jnp.maximum(a2 * u + d2, 0.0)
    work[...] = sv

    pos = jax.lax.broadcasted_iota(jnp.int32, (_B, _N), 1)
    kl = jax.lax.broadcasted_iota(jnp.int32, (_B, _K), 1)
    vacc = jnp.zeros((_B, _K), jnp.float32)
    iacc = jnp.zeros((_B, _K), jnp.int32)
    for k in range(_K):
        w = work[...]
        m = jnp.max(w, axis=1, keepdims=True)
        cand = jnp.where(w == m, pos, _BIGI)
        j = jnp.min(cand, axis=1, keepdims=True)
        vacc = vacc + jnp.where(kl == k, m, 0.0)
        iacc = iacc + jnp.where(kl == k, j, 0)
        work[...] = jnp.where(pos == j, -jnp.inf, w)
    vals_ref[...] = vacc
    idx_ref[...] = iacc


def _gather_kernel(idx_sref, in_ref, out_ref, strips, sems):
    b = pl.program_id(0)
    cps = []
    for k in range(_K):
        i = idx_sref[b * _K + k]
        cp = pltpu.make_async_copy(
            in_ref.at[b, :, pl.ds((i // 128) * 128, 128)], strips.at[k],
            sems.at[k])
        cp.start()
        cps.append(cp)
    lane128 = jax.lax.broadcasted_iota(jnp.int32, (_C1, 128), 1)
    kl = jax.lax.broadcasted_iota(jnp.int32, (_C1, _K), 1)
    acc = jnp.zeros((_C1, _K), jnp.float32)
    for k in range(_K):
        cps[k].wait()
        col = jax.lax.rem(idx_sref[b * _K + k], 128)
        v = jnp.sum(jnp.where(lane128 == col, strips[k], 0.0), axis=1)
        acc = acc + jnp.where(kl == k, v[:, None], 0.0)
    out_ref[0] = acc


def _fold_bn(ssum, ssq, g, be, eps=_EPS):
    nn = jnp.float32(_B * _N)
    m = ssum / nn
    v = ssq / nn - m * m
    a = g * jax.lax.rsqrt(v + eps)
    d = be - a * m
    return a, d


@jax.jit
def kernel(sortvec, input, W0, b0, W1, b1, W2, b2, g0, be0, g1, be1, g2, be2):
    s = sortvec[:, 0, :, :]
    w0b = W0[:, 0, :, 0].astype(jnp.bfloat16)
    w1b = W1[:, :, 0, 0].astype(jnp.bfloat16)
    w2b = jnp.zeros((8, _C2), jnp.bfloat16).at[0].set(
        W2[:, :, 0, 0][0].astype(jnp.bfloat16))
    b0c = b0[:, None]
    b1c = b1[:, None]

    seq2 = pltpu.CompilerParams(dimension_semantics=("arbitrary", "arbitrary"))
    nchunks = _N // _CN
    s_spec = pl.BlockSpec((1, _NF, _CN), lambda b, c: (b, 0, c))
    full = lambda shp: pl.BlockSpec(shp, lambda b, c: tuple(0 for _ in shp))

    stats1, sbf = pl.pallas_call(
        _stats_kernel_a,
        grid=(_B, nchunks),
        in_specs=[s_spec, full((_C1, _NF)), full((_C1, 1))],
        out_specs=[full((_C1, 128)), s_spec],
        out_shape=[jax.ShapeDtypeStruct((_C1, 128), jnp.float32),
                   jax.ShapeDtypeStruct((_B, _NF, _N), jnp.bfloat16)],
        compiler_params=seq2,
    )(s, w0b, b0c)
    a0, d0 = _fold_bn(stats1[:, 0], stats1[:, 1], g0, be0)
    aff0 = jnp.stack([a0, d0, b0], axis=1)

    stats2 = pl.pallas_call(
        _stats_kernel_b,
        grid=(_B, nchunks),
        in_specs=[s_spec, full((_C1, _NF)), full((_C1, 3)),
                  full((_C2, _C1)), full((_C2, 1))],
        out_specs=full((_C2, 128)),
        out_shape=jax.ShapeDtypeStruct((_C2, 128), jnp.float32),
        compiler_params=seq2,
    )(sbf, w0b, aff0, w1b, b1c)
    a1, d1 = _fold_bn(stats2[:, 0], stats2[:, 1], g1, be1)
    aff1 = jnp.stack([a1, d1, b1], axis=1)

    u = pl.pallas_call(
        _score_kernel_c,
        grid=(_B, nchunks),
        in_specs=[s_spec, full((_C1, _NF)), full((_C1, 3)),
                  full((_C2, _C1)), full((_C2, 1)), full((_C2, 3)),
                  full((8, _C2)), pl.BlockSpec(memory_space=pltpu.SMEM)],
        out_specs=pl.BlockSpec((1, 1, _CN), lambda b, c: (b, 0, c)),
        out_shape=jax.ShapeDtypeStruct((_B, 1, _N), jnp.float32),
        compiler_params=seq2,
    )(sbf, w0b, aff0, w1b, b1c, aff1, w2b, b2)

    gb = jnp.stack([g2[0], be2[0]])
    vals, idx = pl.pallas_call(
        _topk_kernel,
        in_specs=[pl.BlockSpec(memory_space=pltpu.VMEM),
                  pl.BlockSpec(memory_space=pltpu.SMEM)],
        out_specs=[pl.BlockSpec(memory_space=pltpu.VMEM),
                   pl.BlockSpec(memory_space=pltpu.VMEM)],
        out_shape=[jax.ShapeDtypeStruct((_B, _K), jnp.float32),
                   jax.ShapeDtypeStruct((_B, _K), jnp.int32)],
        scratch_shapes=[pltpu.VMEM((_B, _N), jnp.float32)],
    )(u, gb)

    grid_spec = pltpu.PrefetchScalarGridSpec(
        num_scalar_prefetch=1,
        grid=(_B,),
        in_specs=[pl.BlockSpec(memory_space=pltpu.MemorySpace.HBM)],
        out_specs=pl.BlockSpec((1, _C1, _K), lambda b, iref: (b, 0, 0)),
        scratch_shapes=[pltpu.VMEM((_K, _C1, 128), jnp.float32),
                        pltpu.SemaphoreType.DMA((_K,))],
    )
    sorted_input = pl.pallas_call(
        _gather_kernel,
        grid_spec=grid_spec,
        out_shape=jax.ShapeDtypeStruct((_B, _C1, _K), jnp.float32),
    )(idx.reshape(-1), input)

    feat = jnp.concatenate([sorted_input, vals[:, None, :]], axis=1)
    return (feat, idx)
